# baseline (device time: 66397 ns/iter reference)
import jax
import jax.numpy as jnp
from jax import lax
from jax.experimental import pallas as pl
from jax.experimental.pallas import tpu as pltpu

N_DEV = 4
N_LOCAL_E = 4


def kernel(x, router_W, route_idx, expert_W, shared_W):
    rows, d_model = x.shape
    d_ff = expert_W.shape[2]
    blk = rows // N_DEV

    def body(x_ref, rw_ref, idx_ref, ew_ref, sw_ref, out_ref,
             send_buf, comm_buf, send_sems, recv_sems):
        my = lax.axis_index("i")

        barrier = pltpu.get_barrier_semaphore()
        for p in range(1, N_DEV):
            pl.semaphore_signal(
                barrier, inc=1,
                device_id=((my + p) % N_DEV,),
                device_id_type=pl.DeviceIdType.MESH,
            )
        pl.semaphore_wait(barrier, N_DEV - 1)

        def block_contrib(dest):
            row_sl = pl.ds(dest * blk, blk)
            xb = x_ref[row_sl, :]
            scores = jnp.dot(xb, rw_ref[:, :], preferred_element_type=jnp.float32)
            scores = scores - jnp.max(scores, axis=1, keepdims=True)
            probs = jnp.exp(scores)
            probs = probs / jnp.sum(probs, axis=1, keepdims=True)
            route = idx_ref[row_sl, :]
            onehot = lax.broadcasted_iota(jnp.int32, probs.shape, 1) == route
            gate = jnp.sum(jnp.where(onehot, probs, 0.0), axis=1, keepdims=True)
            acc = jnp.zeros((blk, d_ff), jnp.float32)
            for j in range(N_LOCAL_E):
                g = jnp.dot(xb, ew_ref[j, :, :], preferred_element_type=jnp.float32)
                w = jnp.where(route == my * N_LOCAL_E + j, gate, 0.0)
                acc = acc + w * g
            return acc

        rdmas = []
        for s in range(1, N_DEV):
            dest = (my + s) % N_DEV
            send_buf[s - 1, :, :] = block_contrib(dest)
            rdma = pltpu.make_async_remote_copy(
                src_ref=send_buf.at[s - 1],
                dst_ref=comm_buf.at[s - 1],
                send_sem=send_sems.at[s - 1],
                recv_sem=recv_sems.at[s - 1],
                device_id=(dest,),
                device_id_type=pl.DeviceIdType.MESH,
            )
            rdma.start()
            rdmas.append(rdma)

        own = block_contrib(my)
        xb = x_ref[pl.ds(my * blk, blk), :]
        own = own + jnp.dot(xb, sw_ref[:, :], preferred_element_type=jnp.float32)

        for s in range(1, N_DEV):
            rdmas[s - 1].wait_recv()
            own = own + comm_buf[s - 1, :, :]
        out_ref[:, :] = own

        for r in rdmas:
            r.wait_send()

    return pl.pallas_call(
        body,
        out_shape=jax.ShapeDtypeStruct((blk, d_ff), jnp.float32),
        in_specs=[pl.BlockSpec(memory_space=pltpu.VMEM)] * 5,
        out_specs=pl.BlockSpec(memory_space=pltpu.VMEM),
        scratch_shapes=[
            pltpu.VMEM((N_DEV - 1, blk, d_ff), jnp.float32),
            pltpu.VMEM((N_DEV - 1, blk, d_ff), jnp.float32),
            pltpu.SemaphoreType.DMA((N_DEV - 1,)),
            pltpu.SemaphoreType.DMA((N_DEV - 1,)),
        ],
        compiler_params=pltpu.CompilerParams(collective_id=0),
    )(x, router_W, route_idx, expert_W, shared_W)


# device time: 44058 ns/iter; 1.5070x vs baseline; 1.5070x over previous
import jax
import jax.numpy as jnp
from jax import lax
from jax.experimental import pallas as pl
from jax.experimental.pallas import tpu as pltpu

N_DEV = 4
N_LOCAL_E = 4


def kernel(x, router_W, route_idx, expert_W, shared_W):
    rows, d_model = x.shape
    d_ff = expert_W.shape[2]
    blk = rows // N_DEV

    def body(x_ref, rw_ref, idx_ref, ew_ref, sw_ref, out_ref,
             send_buf, comm_buf, send_sems, recv_sems):
        my = lax.axis_index("i")

        barrier = pltpu.get_barrier_semaphore()
        for p in range(1, N_DEV):
            pl.semaphore_signal(
                barrier, inc=1,
                device_id=((my + p) % N_DEV,),
                device_id_type=pl.DeviceIdType.MESH,
            )
        pl.semaphore_wait(barrier, N_DEV - 1)

        def block_contrib(dest):
            row_sl = pl.ds(dest * blk, blk)
            xb = x_ref[row_sl, :]
            scores = jnp.dot(xb, rw_ref[:, :], preferred_element_type=jnp.float32)
            scores = scores - jnp.max(scores, axis=1, keepdims=True)
            probs = jnp.exp(scores)
            probs = probs / jnp.sum(probs, axis=1, keepdims=True)
            route = idx_ref[row_sl, :]
            onehot = lax.broadcasted_iota(jnp.int32, probs.shape, 1) == route
            gate = jnp.sum(jnp.where(onehot, probs, 0.0), axis=1, keepdims=True)
            xb16 = xb.astype(jnp.bfloat16)
            acc = jnp.zeros((blk, d_ff), jnp.float32)
            for j in range(N_LOCAL_E):
                g = jnp.dot(xb16, ew_ref[j, :, :].astype(jnp.bfloat16),
                            preferred_element_type=jnp.float32)
                w = jnp.where(route == my * N_LOCAL_E + j, gate, 0.0)
                acc = acc + w * g
            return acc

        rdmas = []
        for s in range(1, N_DEV):
            dest = (my + s) % N_DEV
            send_buf[s - 1, :, :] = block_contrib(dest).astype(jnp.bfloat16)
            rdma = pltpu.make_async_remote_copy(
                src_ref=send_buf.at[s - 1],
                dst_ref=comm_buf.at[s - 1],
                send_sem=send_sems.at[s - 1],
                recv_sem=recv_sems.at[s - 1],
                device_id=(dest,),
                device_id_type=pl.DeviceIdType.MESH,
            )
            rdma.start()
            rdmas.append(rdma)

        own = block_contrib(my)
        xb = x_ref[pl.ds(my * blk, blk), :].astype(jnp.bfloat16)
        own = own + jnp.dot(xb, sw_ref[:, :].astype(jnp.bfloat16),
                            preferred_element_type=jnp.float32)

        for s in range(1, N_DEV):
            rdmas[s - 1].wait_recv()
            own = own + comm_buf[s - 1, :, :].astype(jnp.float32)
        out_ref[:, :] = own

        for r in rdmas:
            r.wait_send()

    return pl.pallas_call(
        body,
        out_shape=jax.ShapeDtypeStruct((blk, d_ff), jnp.float32),
        in_specs=[pl.BlockSpec(memory_space=pltpu.VMEM)] * 5,
        out_specs=pl.BlockSpec(memory_space=pltpu.VMEM),
        scratch_shapes=[
            pltpu.VMEM((N_DEV - 1, blk, d_ff), jnp.bfloat16),
            pltpu.VMEM((N_DEV - 1, blk, d_ff), jnp.bfloat16),
            pltpu.SemaphoreType.DMA((N_DEV - 1,)),
            pltpu.SemaphoreType.DMA((N_DEV - 1,)),
        ],
        compiler_params=pltpu.CompilerParams(collective_id=0),
    )(x, router_W, route_idx, expert_W, shared_W)
